# baseline (speedup 1.0000x reference)
import jax
import jax.numpy as jnp
from jax import lax
from jax.experimental import pallas as pl
from jax.experimental.pallas import tpu as pltpu
from jax.experimental.pallas import tpu_sc as plsc

N_USERS = 100000
EMB_DIM = 32
N_EDGES = 1600000
HALF = 16
NC = 2
NS = 16
LANES = 16

EPT_RAW = N_EDGES // NS
CHUNK = 512
NCHUNK = 196
EPT = CHUNK * NCHUNK
EPTOT = EPT * NS
SUB = 128
NSUB = CHUNK // SUB
R2D_PER_TILE = EPT // SUB
R2D_TOTAL = EPTOT // SUB

NCH = 512
N_FULL = N_USERS // NCH
NTAIL = N_USERS - N_FULL * NCH
NTAIL_OFF = N_FULL * NCH
MCH = 128
M_FULL = N_USERS // MCH
MTAIL = N_USERS - M_FULL * MCH
MTAIL_OFF = M_FULL * MCH

THIRD = 1.0 / 3.0


def _rsqrt16(x):
    xi = lax.bitcast_convert_type(x, jnp.int32)
    yi = jnp.int32(0x5F3759DF) - (xi >> 1)
    y = lax.bitcast_convert_type(yi, jnp.float32)
    for _ in range(3):
        y = y * (1.5 - 0.5 * x * y * y)
    return jnp.where(x > 0.0, y, 0.0)


def _body(x_hbm, rowx_hbm, col2_hbm, w_hbm, out_hbm,
          acc, dinv, x1_hbm,
          rowb, colb, G, drb, dcb, wb, sem, semD):
    c = lax.axis_index("c")
    s = lax.axis_index("s")
    zero16 = jnp.zeros((LANES,), jnp.float32)

    def _zero_wb():
        @pl.loop(0, CHUNK // LANES)
        def _(i):
            wb[pl.ds(i * LANES, LANES)] = zero16

    def _zero_g():
        @pl.loop(0, CHUNK)
        def _(i):
            G[i, :] = zero16

    _zero_wb()

    @pl.loop(s, N_FULL, step=NS)
    def _(i):
        pltpu.sync_copy(wb, dinv.at[pl.ds(i * NCH, NCH)])

    @pl.when(s == NS - 1)
    def _():
        pltpu.sync_copy(wb.at[pl.ds(0, NTAIL)],
                        dinv.at[pl.ds(NTAIL_OFF, NTAIL)])

    plsc.subcore_barrier()

    @pl.loop(0, NCHUNK)
    def _(co):
        off = s * EPT + co * CHUNK
        r2 = s * R2D_PER_TILE + co * NSUB
        pltpu.sync_copy(col2_hbm.at[pl.ds(r2, NSUB)], colb)
        pltpu.sync_copy(w_hbm.at[pl.ds(off, CHUNK)], wb)
        for j in range(NSUB):
            pltpu.sync_copy(wb.at[pl.ds(j * SUB, SUB)],
                            dinv.at[colb.at[j]], add=True)

    plsc.subcore_barrier()

    def _dinv_chunk(off, n):
        pltpu.sync_copy(dinv.at[pl.ds(off, n)], wb.at[pl.ds(0, n)])

        @pl.loop(0, n // LANES)
        def _(g):
            x = wb[pl.ds(g * LANES, LANES)]
            drb[pl.ds(g * LANES, LANES)] = _rsqrt16(x)

        pltpu.sync_copy(drb.at[pl.ds(0, n)], dinv.at[pl.ds(off, n)])
        pltpu.sync_copy(drb.at[pl.ds(0, n)], dinv.at[pl.ds(N_USERS + off, n)])

    @pl.loop(s, N_FULL, step=NS)
    def _(i):
        _dinv_chunk(i * NCH, NCH)

    @pl.when(s == NS - 1)
    def _():
        _dinv_chunk(NTAIL_OFF, NTAIL)

    plsc.subcore_barrier()


    def _zero_acc():
        _zero_g()

        @pl.loop(s, N_FULL, step=NS)
        def _(i):
            pltpu.sync_copy(G, acc.at[pl.ds(i * NCH, NCH)])

        @pl.when(s == NS - 1)
        def _():
            pltpu.sync_copy(G.at[pl.ds(0, NTAIL)],
                            acc.at[pl.ds(NTAIL_OFF, NTAIL)])

    def _layer(src_hbm):
        @pl.loop(0, NCHUNK)
        def _(co):
            off = s * EPT + co * CHUNK
            r2 = s * R2D_PER_TILE + co * NSUB
            pltpu.sync_copy(rowx_hbm.at[pl.ds(c * R2D_TOTAL + r2, NSUB)], rowb)
            pltpu.sync_copy(col2_hbm.at[pl.ds(r2, NSUB)], colb)
            pltpu.sync_copy(w_hbm.at[pl.ds(off, CHUNK)], wb)
            descs = []
            for j in range(NSUB):
                descs.append(pltpu.async_copy(
                    dinv.at[rowb.at[j]], drb.at[pl.ds(j * SUB, SUB)], semD))
                descs.append(pltpu.async_copy(
                    dinv.at[colb.at[j]], dcb.at[pl.ds(j * SUB, SUB)], semD))
                descs.append(pltpu.async_copy(
                    src_hbm.at[rowb.at[j]], G.at[pl.ds(j * SUB, SUB)], sem))
            for d in descs:
                d.wait()

            @pl.loop(0, CHUNK // LANES)
            def _(g):
                b = g * LANES
                nv16 = (drb[pl.ds(b, LANES)]
                        * wb[pl.ds(b, LANES)]
                        * dcb[pl.ds(b, LANES)])
                for i in range(LANES):
                    r = b + i
                    G[r, :] = G[r, :] * nv16[i]

            for j in range(NSUB):
                pltpu.sync_copy(G.at[pl.ds(j * SUB, SUB)],
                                acc.at[colb.at[j]], add=True)

    @pl.loop(s, N_FULL, step=NS)
    def _(i):
        off = c * N_USERS + i * NCH
        pltpu.sync_copy(x_hbm.at[pl.ds(off, NCH)], x1_hbm.at[pl.ds(off, NCH)])

    @pl.when(s == NS - 1)
    def _():
        off = c * N_USERS + NTAIL_OFF
        pltpu.sync_copy(x_hbm.at[pl.ds(off, NTAIL)],
                        x1_hbm.at[pl.ds(off, NTAIL)])

    plsc.subcore_barrier()

    @pl.loop(0, 2)
    def _(layer_i):
        _zero_acc()
        plsc.subcore_barrier()
        _layer(x1_hbm)
        plsc.subcore_barrier()

        @pl.when(layer_i == 0)
        def _():
            @pl.loop(s, N_FULL, step=NS)
            def _(i):
                off = i * NCH
                pltpu.sync_copy(acc.at[pl.ds(off, NCH)],
                                x1_hbm.at[pl.ds(c * N_USERS + off, NCH)])

            @pl.when(s == NS - 1)
            def _():
                pltpu.sync_copy(
                    acc.at[pl.ds(NTAIL_OFF, NTAIL)],
                    x1_hbm.at[pl.ds(c * N_USERS + NTAIL_OFF, NTAIL)])

            plsc.subcore_barrier()

    def _mean_chunk(off, n):
        base = c * N_USERS + off
        pltpu.sync_copy(x_hbm.at[pl.ds(base, n)], G.at[pl.ds(0, n)])
        pltpu.sync_copy(x1_hbm.at[pl.ds(base, n)], G.at[pl.ds(MCH, n)])
        pltpu.sync_copy(acc.at[pl.ds(off, n)], G.at[pl.ds(2 * MCH, n)])

        @pl.loop(0, n)
        def _(i):
            G[i, :] = (G[i, :] + G[MCH + i, :] + G[2 * MCH + i, :]) \
                * jnp.float32(THIRD)

        pltpu.sync_copy(G.at[pl.ds(0, n)], out_hbm.at[pl.ds(base, n)])

    @pl.loop(s, M_FULL, step=NS)
    def _(i):
        _mean_chunk(i * MCH, MCH)

    @pl.when(s == NS - 1)
    def _():
        _mean_chunk(MTAIL_OFF, MTAIL)


def _make_kernel():
    mesh = plsc.VectorSubcoreMesh(core_axis_name="c", subcore_axis_name="s")
    return pl.kernel(
        _body,
        out_type=jax.ShapeDtypeStruct((NC * N_USERS, HALF), jnp.float32),
        mesh=mesh,
        scratch_types=[
            pltpu.VMEM_SHARED((N_USERS, HALF), jnp.float32),
            pltpu.VMEM_SHARED((2 * N_USERS,), jnp.float32),
            pltpu.HBM((NC * N_USERS, HALF), jnp.float32),
            pltpu.VMEM((NSUB, SUB), jnp.int32),
            pltpu.VMEM((NSUB, SUB), jnp.int32),
            pltpu.VMEM((CHUNK, HALF), jnp.float32),
            pltpu.VMEM((CHUNK,), jnp.float32),
            pltpu.VMEM((CHUNK,), jnp.float32),
            pltpu.VMEM((CHUNK,), jnp.float32),
            pltpu.SemaphoreType.DMA,
            pltpu.SemaphoreType.DMA,
        ],
        compiler_params=pltpu.CompilerParams(use_tc_tiling_on_sc=False),
    )


@jax.jit
def kernel(edge_index, edge_weight, user_emb):
    row = edge_index[0].reshape(NS, EPT_RAW)
    col = edge_index[1].reshape(NS, EPT_RAW)
    w = edge_weight.reshape(NS, EPT_RAW)
    pad = EPT - EPT_RAW
    rowp = jnp.pad(row, ((0, 0), (0, pad))).reshape(-1)
    colp = jnp.pad(col, ((0, 0), (0, pad))).reshape(-1)
    wp = jnp.pad(w, ((0, 0), (0, pad))).reshape(-1)
    rowx = jnp.concatenate([rowp, rowp + N_USERS]).reshape(NC * R2D_TOTAL, SUB)
    col2 = colp.reshape(R2D_TOTAL, SUB)
    xh = user_emb.reshape(N_USERS, NC, HALF).transpose(1, 0, 2).reshape(
        NC * N_USERS, HALF)
    social = _make_kernel()(xh, rowx, col2, wp)
    return social.reshape(NC, N_USERS, HALF).transpose(1, 0, 2).reshape(
        N_USERS, EMB_DIM)


# speedup vs baseline: 1.4148x; 1.4148x over previous
import jax
import jax.numpy as jnp
from jax import lax
from jax.experimental import pallas as pl
from jax.experimental.pallas import tpu as pltpu
from jax.experimental.pallas import tpu_sc as plsc

N_USERS = 100000
EMB_DIM = 32
N_EDGES = 1600000
HALF = 16
NC = 2
NS = 16
LANES = 16

EPT_RAW = N_EDGES // NS
CHUNK = 256
NCHUNK = 396
EPT = CHUNK * NCHUNK
EPTOT = EPT * NS
SUB = 128
NSUB = CHUNK // SUB
R2D_PER_TILE = EPT // SUB
R2D_TOTAL = EPTOT // SUB

NLIN = 4
NG = 3
UNROLL = 12
NIT = NCHUNK // UNROLL
DUNROLL = 4
DNIT = NCHUNK // DUNROLL

ZCH = 256
Z_FULL = N_USERS // ZCH
ZTAIL = N_USERS - Z_FULL * ZCH
ZTAIL_OFF = Z_FULL * ZCH
WCH = 512
W_FULL = N_USERS // WCH
WTAIL = N_USERS - W_FULL * WCH
WTAIL_OFF = W_FULL * WCH
MCH = 128
M_FULL = N_USERS // MCH
MTAIL = N_USERS - M_FULL * MCH
MTAIL_OFF = M_FULL * MCH

THIRD = 1.0 / 3.0


def _rsqrt16(x):
    xi = lax.bitcast_convert_type(x, jnp.int32)
    yi = jnp.int32(0x5F3759DF) - (xi >> 1)
    y = lax.bitcast_convert_type(yi, jnp.float32)
    for _ in range(3):
        y = y * (1.5 - 0.5 * x * y * y)
    return jnp.where(x > 0.0, y, 0.0)


def _body(x_hbm, rowx_hbm, col2_hbm, w_hbm, out_hbm,
          acc, dinv, x1_hbm,
          rowb0, rowb1, rowb2, rowb3, colb0, colb1, colb2, colb3,
          wb, drb, dcb, G,
          semL, semG, semD, semS):
    rowbs = [rowb0, rowb1, rowb2, rowb3]
    colbs = [colb0, colb1, colb2, colb3]
    c = lax.axis_index("c")
    s = lax.axis_index("s")
    zero16 = jnp.zeros((LANES,), jnp.float32)

    @pl.loop(0, CHUNK // LANES)
    def _(i):
        wb[0, pl.ds(i * LANES, LANES)] = zero16

    @pl.loop(s, Z_FULL, step=NS)
    def _(i):
        pltpu.sync_copy(wb.at[0], dinv.at[pl.ds(i * ZCH, ZCH)])

    @pl.when(s == NS - 1)
    def _():
        pltpu.sync_copy(wb.at[0].at[pl.ds(0, ZTAIL)],
                        dinv.at[pl.ds(ZTAIL_OFF, ZTAIL)])

    plsc.subcore_barrier()

    def _deg_lin(co, k):
        off = s * EPT + co * CHUNK
        r2 = s * R2D_PER_TILE + co * NSUB
        pltpu.async_copy(col2_hbm.at[pl.ds(r2, NSUB)], colbs[k], semL.at[k])
        pltpu.async_copy(w_hbm.at[pl.ds(off, CHUNK)], wb.at[k], semL.at[k])

    def _deg_lin_wait(k):
        pltpu.make_async_copy(col2_hbm.at[pl.ds(0, NSUB)], colbs[k],
                              semL.at[k]).wait()
        pltpu.make_async_copy(w_hbm.at[pl.ds(0, CHUNK)], wb.at[k],
                              semL.at[k]).wait()

    _deg_lin(0, 0)
    _deg_lin(1, 1)

    @pl.loop(0, DNIT)
    def _(it):
        for u in range(DUNROLL):
            co = it * DUNROLL + u

            if u < 2:
                _deg_lin(co + 2, (u + 2) % NLIN)
            else:
                @pl.when(it < DNIT - 1)
                def _(co=co, u=u):
                    _deg_lin(co + 2, (u + 2) % NLIN)

            _deg_lin_wait(u)
            for j in range(NSUB):
                pltpu.sync_copy(wb.at[u].at[pl.ds(j * SUB, SUB)],
                                dinv.at[colbs[u].at[j]], add=True)

    plsc.subcore_barrier()

    def _dinv_chunk(off, n):
        pltpu.sync_copy(dinv.at[pl.ds(off, n)], wb.at[0].at[pl.ds(0, n)])

        @pl.loop(0, n // LANES)
        def _(g):
            x = wb[0, pl.ds(g * LANES, LANES)]
            drb[0, pl.ds(g * LANES, LANES)] = _rsqrt16(x)

        pltpu.sync_copy(drb.at[0].at[pl.ds(0, n)], dinv.at[pl.ds(off, n)])
        pltpu.sync_copy(drb.at[0].at[pl.ds(0, n)],
                        dinv.at[pl.ds(N_USERS + off, n)])

    @pl.loop(s, Z_FULL, step=NS)
    def _(i):
        _dinv_chunk(i * ZCH, ZCH)

    @pl.when(s == NS - 1)
    def _():
        _dinv_chunk(ZTAIL_OFF, ZTAIL)

    @pl.loop(s, W_FULL, step=NS)
    def _(i):
        off = c * N_USERS + i * WCH
        pltpu.sync_copy(x_hbm.at[pl.ds(off, WCH)], x1_hbm.at[pl.ds(off, WCH)])

    @pl.when(s == NS - 1)
    def _():
        off = c * N_USERS + WTAIL_OFF
        pltpu.sync_copy(x_hbm.at[pl.ds(off, WTAIL)],
                        x1_hbm.at[pl.ds(off, WTAIL)])

    plsc.subcore_barrier()

    def _zero_acc():
        @pl.loop(0, CHUNK)
        def _(i):
            G[0, i, :] = zero16

        @pl.loop(s, Z_FULL, step=NS)
        def _(i):
            pltpu.sync_copy(G.at[0], acc.at[pl.ds(i * ZCH, ZCH)])

        @pl.when(s == NS - 1)
        def _():
            pltpu.sync_copy(G.at[0].at[pl.ds(0, ZTAIL)],
                            acc.at[pl.ds(ZTAIL_OFF, ZTAIL)])

    def _lin(co, k):
        off = s * EPT + co * CHUNK
        r2 = s * R2D_PER_TILE + co * NSUB
        pltpu.async_copy(rowx_hbm.at[pl.ds(c * R2D_TOTAL + r2, NSUB)],
                         rowbs[k], semL.at[k])
        pltpu.async_copy(col2_hbm.at[pl.ds(r2, NSUB)], colbs[k], semL.at[k])
        pltpu.async_copy(w_hbm.at[pl.ds(off, CHUNK)], wb.at[k], semL.at[k])

    def _lin_wait(k):
        pltpu.make_async_copy(col2_hbm.at[pl.ds(0, NSUB)], rowbs[k],
                              semL.at[k]).wait()
        pltpu.make_async_copy(col2_hbm.at[pl.ds(0, NSUB)], colbs[k],
                              semL.at[k]).wait()
        pltpu.make_async_copy(w_hbm.at[pl.ds(0, CHUNK)], wb.at[k],
                              semL.at[k]).wait()

    def _fire_gathers(kL, kG):
        for j in range(NSUB):
            pltpu.async_copy(dinv.at[rowbs[kL].at[j]],
                             drb.at[kG].at[pl.ds(j * SUB, SUB)], semD.at[kG])
            pltpu.async_copy(dinv.at[colbs[kL].at[j]],
                             dcb.at[kG].at[pl.ds(j * SUB, SUB)], semD.at[kG])
            pltpu.async_copy(x1_hbm.at[rowbs[kL].at[j]],
                             G.at[kG].at[pl.ds(j * SUB, SUB)], semG.at[kG])

    def _gather_wait(kL, kG):
        for j in range(NSUB):
            pltpu.make_async_copy(dinv.at[rowbs[kL].at[j]],
                                  drb.at[kG].at[pl.ds(j * SUB, SUB)],
                                  semD.at[kG]).wait()
            pltpu.make_async_copy(dinv.at[colbs[kL].at[j]],
                                  dcb.at[kG].at[pl.ds(j * SUB, SUB)],
                                  semD.at[kG]).wait()
            pltpu.make_async_copy(x1_hbm.at[rowbs[kL].at[j]],
                                  G.at[kG].at[pl.ds(j * SUB, SUB)],
                                  semG.at[kG]).wait()

    def _scat_drain(kL, kG):
        for j in range(NSUB):
            pltpu.make_async_copy(G.at[kG].at[pl.ds(j * SUB, SUB)],
                                  acc.at[colbs[kL].at[j]], semS.at[kG]).wait()

    @pl.loop(0, 2)
    def _(layer_i):
        _zero_acc()
        plsc.subcore_barrier()

        _lin(0, 0)
        _lin(1, 1)
        _lin_wait(0)
        _fire_gathers(0, 0)

        @pl.loop(0, NIT)
        def _(it):
            for u in range(UNROLL):
                co = it * UNROLL + u
                sl = u % NLIN
                nsl = (u + 1) % NLIN
                psl = (u + 2) % NLIN
                kG = u % NG
                nkG = (u + 1) % NG
                dL = (u + 2) % NLIN
                dG = (u + 1) % NG

                if u >= 2:
                    _scat_drain(dL, dG)
                else:
                    @pl.when(it >= 1)
                    def _(dL=dL, dG=dG):
                        _scat_drain(dL, dG)

                if u < UNROLL - 2:
                    _lin(co + 2, psl)
                else:
                    @pl.when(it < NIT - 1)
                    def _(co=co, psl=psl):
                        _lin(co + 2, psl)

                _gather_wait(sl, kG)

                if u < UNROLL - 1:
                    _lin_wait(nsl)
                    _fire_gathers(nsl, nkG)
                else:
                    @pl.when(it < NIT - 1)
                    def _(nsl=nsl, nkG=nkG):
                        _lin_wait(nsl)
                        _fire_gathers(nsl, nkG)

                @pl.loop(0, CHUNK // LANES)
                def _(g, kG=kG, sl=sl):
                    b = g * LANES
                    nv16 = (drb[kG, pl.ds(b, LANES)]
                            * wb[sl, pl.ds(b, LANES)]
                            * dcb[kG, pl.ds(b, LANES)])
                    for i in range(LANES):
                        r = b + i
                        G[kG, r, :] = G[kG, r, :] * nv16[i]

                for j in range(NSUB):
                    pltpu.async_copy(G.at[kG].at[pl.ds(j * SUB, SUB)],
                                     acc.at[colbs[sl].at[j]], semS.at[kG],
                                     add=True)

        _scat_drain((NCHUNK - 2) % NLIN, (NCHUNK - 2) % NG)
        _scat_drain((NCHUNK - 1) % NLIN, (NCHUNK - 1) % NG)

        plsc.subcore_barrier()

        @pl.when(layer_i == 0)
        def _():
            @pl.loop(s, W_FULL, step=NS)
            def _(i):
                off = i * WCH
                pltpu.sync_copy(acc.at[pl.ds(off, WCH)],
                                x1_hbm.at[pl.ds(c * N_USERS + off, WCH)])

            @pl.when(s == NS - 1)
            def _():
                pltpu.sync_copy(
                    acc.at[pl.ds(WTAIL_OFF, WTAIL)],
                    x1_hbm.at[pl.ds(c * N_USERS + WTAIL_OFF, WTAIL)])

            plsc.subcore_barrier()

    def _mean_chunk(off, n):
        base = c * N_USERS + off
        pltpu.sync_copy(x_hbm.at[pl.ds(base, n)], G.at[0].at[pl.ds(0, n)])
        pltpu.sync_copy(x1_hbm.at[pl.ds(base, n)], G.at[1].at[pl.ds(0, n)])
        pltpu.sync_copy(acc.at[pl.ds(off, n)], G.at[2].at[pl.ds(0, n)])

        @pl.loop(0, n)
        def _(i):
            G[0, i, :] = (G[0, i, :] + G[1, i, :] + G[2, i, :]) \
                * jnp.float32(THIRD)

        pltpu.sync_copy(G.at[0].at[pl.ds(0, n)], out_hbm.at[pl.ds(base, n)])

    @pl.loop(s, M_FULL, step=NS)
    def _(i):
        _mean_chunk(i * MCH, MCH)

    @pl.when(s == NS - 1)
    def _():
        _mean_chunk(MTAIL_OFF, MTAIL)


def _make_kernel():
    mesh = plsc.VectorSubcoreMesh(core_axis_name="c", subcore_axis_name="s")
    return pl.kernel(
        _body,
        out_type=jax.ShapeDtypeStruct((NC * N_USERS, HALF), jnp.float32),
        mesh=mesh,
        scratch_types=[
            pltpu.VMEM_SHARED((N_USERS, HALF), jnp.float32),
            pltpu.VMEM_SHARED((2 * N_USERS,), jnp.float32),
            pltpu.HBM((NC * N_USERS, HALF), jnp.float32),
            pltpu.VMEM((NSUB, SUB), jnp.int32),
            pltpu.VMEM((NSUB, SUB), jnp.int32),
            pltpu.VMEM((NSUB, SUB), jnp.int32),
            pltpu.VMEM((NSUB, SUB), jnp.int32),
            pltpu.VMEM((NSUB, SUB), jnp.int32),
            pltpu.VMEM((NSUB, SUB), jnp.int32),
            pltpu.VMEM((NSUB, SUB), jnp.int32),
            pltpu.VMEM((NSUB, SUB), jnp.int32),
            pltpu.VMEM((NLIN, CHUNK), jnp.float32),
            pltpu.VMEM((NG, CHUNK), jnp.float32),
            pltpu.VMEM((NG, CHUNK), jnp.float32),
            pltpu.VMEM((NG, CHUNK, HALF), jnp.float32),
            pltpu.SemaphoreType.DMA((NLIN,)),
            pltpu.SemaphoreType.DMA((NG,)),
            pltpu.SemaphoreType.DMA((NG,)),
            pltpu.SemaphoreType.DMA((NG,)),
        ],
        compiler_params=pltpu.CompilerParams(use_tc_tiling_on_sc=False),
    )


@jax.jit
def kernel(edge_index, edge_weight, user_emb):
    row = edge_index[0].reshape(NS, EPT_RAW)
    col = edge_index[1].reshape(NS, EPT_RAW)
    w = edge_weight.reshape(NS, EPT_RAW)
    pad = EPT - EPT_RAW
    rowp = jnp.pad(row, ((0, 0), (0, pad))).reshape(-1)
    colp = jnp.pad(col, ((0, 0), (0, pad))).reshape(-1)
    wp = jnp.pad(w, ((0, 0), (0, pad))).reshape(-1)
    rowx = jnp.concatenate([rowp, rowp + N_USERS]).reshape(NC * R2D_TOTAL, SUB)
    col2 = colp.reshape(R2D_TOTAL, SUB)
    xh = user_emb.reshape(N_USERS, NC, HALF).transpose(1, 0, 2).reshape(
        NC * N_USERS, HALF)
    social = _make_kernel()(xh, rowx, col2, wp)
    return social.reshape(NC, N_USERS, HALF).transpose(1, 0, 2).reshape(
        N_USERS, EMB_DIM)


# speedup vs baseline: 1.4326x; 1.0126x over previous
import jax
import jax.numpy as jnp
from jax import lax
from jax.experimental import pallas as pl
from jax.experimental.pallas import tpu as pltpu
from jax.experimental.pallas import tpu_sc as plsc

N_USERS = 100000
EMB_DIM = 32
N_EDGES = 1600000
HALF = 16
NC = 2
NS = 16
LANES = 16

EPT_RAW = N_EDGES // NS
CHUNK = 256
NCHUNK = 396
EPT = CHUNK * NCHUNK
EPTOT = EPT * NS
SUB = 128
NSUB = CHUNK // SUB
R2D_PER_TILE = EPT // SUB
R2D_TOTAL = EPTOT // SUB

NLIN = 4
NG = 3
UNROLL = 12
NIT = NCHUNK // UNROLL
DUNROLL = 4
DNIT = NCHUNK // DUNROLL

ZCH = 256
Z_FULL = N_USERS // ZCH
ZTAIL = N_USERS - Z_FULL * ZCH
ZTAIL_OFF = Z_FULL * ZCH
WCH = 512
W_FULL = N_USERS // WCH
WTAIL = N_USERS - W_FULL * WCH
WTAIL_OFF = W_FULL * WCH
MCH = 128
M_FULL = N_USERS // MCH
MTAIL = N_USERS - M_FULL * MCH
MTAIL_OFF = M_FULL * MCH

THIRD = 1.0 / 3.0


def _rsqrt16(x):
    xi = lax.bitcast_convert_type(x, jnp.int32)
    yi = jnp.int32(0x5F3759DF) - (xi >> 1)
    y = lax.bitcast_convert_type(yi, jnp.float32)
    for _ in range(3):
        y = y * (1.5 - 0.5 * x * y * y)
    return jnp.where(x > 0.0, y, 0.0)


def _body(x_hbm, rowx_hbm, col2_hbm, w_hbm, out_hbm,
          acc, dinv, x1_hbm,
          rowb0, rowb1, rowb2, rowb3, colb0, colb1, colb2, colb3,
          wb, drb, dcb, G,
          semL, semG, semD, semS):
    rowbs = [rowb0, rowb1, rowb2, rowb3]
    colbs = [colb0, colb1, colb2, colb3]
    c = lax.axis_index("c")
    s = lax.axis_index("s")
    zero16 = jnp.zeros((LANES,), jnp.float32)

    @pl.loop(0, CHUNK // LANES)
    def _(i):
        wb[0, pl.ds(i * LANES, LANES)] = zero16

    @pl.loop(s, Z_FULL, step=NS)
    def _(i):
        pltpu.sync_copy(wb.at[0], dinv.at[pl.ds(i * ZCH, ZCH)])

    @pl.when(s == NS - 1)
    def _():
        pltpu.sync_copy(wb.at[0].at[pl.ds(0, ZTAIL)],
                        dinv.at[pl.ds(ZTAIL_OFF, ZTAIL)])

    plsc.subcore_barrier()

    def _deg_lin(co, k):
        off = s * EPT + co * CHUNK
        r2 = s * R2D_PER_TILE + co * NSUB
        pltpu.async_copy(col2_hbm.at[pl.ds(r2, NSUB)], colbs[k], semL.at[k])
        pltpu.async_copy(w_hbm.at[pl.ds(off, CHUNK)], wb.at[k], semL.at[k])

    def _deg_lin_wait(k):
        pltpu.make_async_copy(col2_hbm.at[pl.ds(0, NSUB)], colbs[k],
                              semL.at[k]).wait()
        pltpu.make_async_copy(w_hbm.at[pl.ds(0, CHUNK)], wb.at[k],
                              semL.at[k]).wait()

    def _deg_scat_drain(p, k):
        for j in range(NSUB):
            pltpu.make_async_copy(wb.at[0].at[pl.ds(j * SUB, SUB)],
                                  dinv.at[colbs[k].at[j]], semS.at[p]).wait()

    _deg_lin(0, 0)
    _deg_lin(1, 1)

    @pl.loop(0, DNIT)
    def _(it):
        for u in range(DUNROLL):
            co = it * DUNROLL + u
            p = u % 2

            if u >= 2:
                _deg_scat_drain(p, (u + 2) % NLIN)
            else:
                @pl.when(it >= 1)
                def _(p=p, u=u):
                    _deg_scat_drain(p, (u + 2) % NLIN)

            if u < 2:
                _deg_lin(co + 2, (u + 2) % NLIN)
            else:
                @pl.when(it < DNIT - 1)
                def _(co=co, u=u):
                    _deg_lin(co + 2, (u + 2) % NLIN)

            _deg_lin_wait(u)
            for j in range(NSUB):
                pltpu.async_copy(wb.at[u].at[pl.ds(j * SUB, SUB)],
                                 dinv.at[colbs[u].at[j]], semS.at[p], add=True)

    _deg_scat_drain(0, 2)
    _deg_scat_drain(1, 3)

    plsc.subcore_barrier()

    def _dinv_chunk(off, n):
        pltpu.sync_copy(dinv.at[pl.ds(off, n)], wb.at[0].at[pl.ds(0, n)])

        @pl.loop(0, n // LANES)
        def _(g):
            x = wb[0, pl.ds(g * LANES, LANES)]
            drb[0, pl.ds(g * LANES, LANES)] = _rsqrt16(x)

        pltpu.sync_copy(drb.at[0].at[pl.ds(0, n)], dinv.at[pl.ds(off, n)])
        pltpu.sync_copy(drb.at[0].at[pl.ds(0, n)],
                        dinv.at[pl.ds(N_USERS + off, n)])

    @pl.loop(s, Z_FULL, step=NS)
    def _(i):
        _dinv_chunk(i * ZCH, ZCH)

    @pl.when(s == NS - 1)
    def _():
        _dinv_chunk(ZTAIL_OFF, ZTAIL)

    @pl.loop(s, W_FULL, step=NS)
    def _(i):
        off = c * N_USERS + i * WCH
        pltpu.sync_copy(x_hbm.at[pl.ds(off, WCH)], x1_hbm.at[pl.ds(off, WCH)])

    @pl.when(s == NS - 1)
    def _():
        off = c * N_USERS + WTAIL_OFF
        pltpu.sync_copy(x_hbm.at[pl.ds(off, WTAIL)],
                        x1_hbm.at[pl.ds(off, WTAIL)])

    plsc.subcore_barrier()

    def _zero_acc():
        @pl.loop(0, CHUNK)
        def _(i):
            G[0, i, :] = zero16

        @pl.loop(s, Z_FULL, step=NS)
        def _(i):
            pltpu.sync_copy(G.at[0], acc.at[pl.ds(i * ZCH, ZCH)])

        @pl.when(s == NS - 1)
        def _():
            pltpu.sync_copy(G.at[0].at[pl.ds(0, ZTAIL)],
                            acc.at[pl.ds(ZTAIL_OFF, ZTAIL)])

    def _lin(co, k):
        off = s * EPT + co * CHUNK
        r2 = s * R2D_PER_TILE + co * NSUB
        pltpu.async_copy(rowx_hbm.at[pl.ds(c * R2D_TOTAL + r2, NSUB)],
                         rowbs[k], semL.at[k])
        pltpu.async_copy(col2_hbm.at[pl.ds(r2, NSUB)], colbs[k], semL.at[k])
        pltpu.async_copy(w_hbm.at[pl.ds(off, CHUNK)], wb.at[k], semL.at[k])

    def _lin_wait(k):
        pltpu.make_async_copy(col2_hbm.at[pl.ds(0, NSUB)], rowbs[k],
                              semL.at[k]).wait()
        pltpu.make_async_copy(col2_hbm.at[pl.ds(0, NSUB)], colbs[k],
                              semL.at[k]).wait()
        pltpu.make_async_copy(w_hbm.at[pl.ds(0, CHUNK)], wb.at[k],
                              semL.at[k]).wait()

    def _fire_gathers(kL, kG):
        for j in range(NSUB):
            pltpu.async_copy(dinv.at[rowbs[kL].at[j]],
                             drb.at[kG].at[pl.ds(j * SUB, SUB)], semD.at[kG])
            pltpu.async_copy(dinv.at[colbs[kL].at[j]],
                             dcb.at[kG].at[pl.ds(j * SUB, SUB)], semD.at[kG])
            pltpu.async_copy(x1_hbm.at[rowbs[kL].at[j]],
                             G.at[kG].at[pl.ds(j * SUB, SUB)], semG.at[kG])

    def _gather_wait(kL, kG):
        for j in range(NSUB):
            pltpu.make_async_copy(dinv.at[rowbs[kL].at[j]],
                                  drb.at[kG].at[pl.ds(j * SUB, SUB)],
                                  semD.at[kG]).wait()
            pltpu.make_async_copy(dinv.at[colbs[kL].at[j]],
                                  dcb.at[kG].at[pl.ds(j * SUB, SUB)],
                                  semD.at[kG]).wait()
            pltpu.make_async_copy(x1_hbm.at[rowbs[kL].at[j]],
                                  G.at[kG].at[pl.ds(j * SUB, SUB)],
                                  semG.at[kG]).wait()

    def _scat_drain(kL, kG):
        for j in range(NSUB):
            pltpu.make_async_copy(G.at[kG].at[pl.ds(j * SUB, SUB)],
                                  acc.at[colbs[kL].at[j]], semS.at[kG]).wait()

    @pl.loop(0, 2)
    def _(layer_i):
        _zero_acc()
        plsc.subcore_barrier()

        _lin(0, 0)
        _lin(1, 1)
        _lin_wait(0)
        _fire_gathers(0, 0)

        @pl.loop(0, NIT)
        def _(it):
            for u in range(UNROLL):
                co = it * UNROLL + u
                sl = u % NLIN
                nsl = (u + 1) % NLIN
                psl = (u + 2) % NLIN
                kG = u % NG
                nkG = (u + 1) % NG
                dL = (u + 2) % NLIN
                dG = (u + 1) % NG

                if u >= 2:
                    _scat_drain(dL, dG)
                else:
                    @pl.when(it >= 1)
                    def _(dL=dL, dG=dG):
                        _scat_drain(dL, dG)

                if u < UNROLL - 2:
                    _lin(co + 2, psl)
                else:
                    @pl.when(it < NIT - 1)
                    def _(co=co, psl=psl):
                        _lin(co + 2, psl)

                _gather_wait(sl, kG)

                if u < UNROLL - 1:
                    _lin_wait(nsl)
                    _fire_gathers(nsl, nkG)
                else:
                    @pl.when(it < NIT - 1)
                    def _(nsl=nsl, nkG=nkG):
                        _lin_wait(nsl)
                        _fire_gathers(nsl, nkG)

                @pl.loop(0, CHUNK // LANES)
                def _(g, kG=kG, sl=sl):
                    b = g * LANES
                    nv16 = (drb[kG, pl.ds(b, LANES)]
                            * wb[sl, pl.ds(b, LANES)]
                            * dcb[kG, pl.ds(b, LANES)])
                    for i in range(LANES):
                        r = b + i
                        G[kG, r, :] = G[kG, r, :] * nv16[i]

                for j in range(NSUB):
                    pltpu.async_copy(G.at[kG].at[pl.ds(j * SUB, SUB)],
                                     acc.at[colbs[sl].at[j]], semS.at[kG],
                                     add=True)

        _scat_drain((NCHUNK - 2) % NLIN, (NCHUNK - 2) % NG)
        _scat_drain((NCHUNK - 1) % NLIN, (NCHUNK - 1) % NG)

        plsc.subcore_barrier()

        @pl.when(layer_i == 0)
        def _():
            @pl.loop(s, W_FULL, step=NS)
            def _(i):
                off = i * WCH
                pltpu.sync_copy(acc.at[pl.ds(off, WCH)],
                                x1_hbm.at[pl.ds(c * N_USERS + off, WCH)])

            @pl.when(s == NS - 1)
            def _():
                pltpu.sync_copy(
                    acc.at[pl.ds(WTAIL_OFF, WTAIL)],
                    x1_hbm.at[pl.ds(c * N_USERS + WTAIL_OFF, WTAIL)])

            plsc.subcore_barrier()

    def _mean_chunk(off, n):
        base = c * N_USERS + off
        pltpu.sync_copy(x_hbm.at[pl.ds(base, n)], G.at[0].at[pl.ds(0, n)])
        pltpu.sync_copy(x1_hbm.at[pl.ds(base, n)], G.at[1].at[pl.ds(0, n)])
        pltpu.sync_copy(acc.at[pl.ds(off, n)], G.at[2].at[pl.ds(0, n)])

        @pl.loop(0, n)
        def _(i):
            G[0, i, :] = (G[0, i, :] + G[1, i, :] + G[2, i, :]) \
                * jnp.float32(THIRD)

        pltpu.sync_copy(G.at[0].at[pl.ds(0, n)], out_hbm.at[pl.ds(base, n)])

    @pl.loop(s, M_FULL, step=NS)
    def _(i):
        _mean_chunk(i * MCH, MCH)

    @pl.when(s == NS - 1)
    def _():
        _mean_chunk(MTAIL_OFF, MTAIL)


def _make_kernel():
    mesh = plsc.VectorSubcoreMesh(core_axis_name="c", subcore_axis_name="s")
    return pl.kernel(
        _body,
        out_type=jax.ShapeDtypeStruct((NC * N_USERS, HALF), jnp.float32),
        mesh=mesh,
        scratch_types=[
            pltpu.VMEM_SHARED((N_USERS, HALF), jnp.float32),
            pltpu.VMEM_SHARED((2 * N_USERS,), jnp.float32),
            pltpu.HBM((NC * N_USERS, HALF), jnp.float32),
            pltpu.VMEM((NSUB, SUB), jnp.int32),
            pltpu.VMEM((NSUB, SUB), jnp.int32),
            pltpu.VMEM((NSUB, SUB), jnp.int32),
            pltpu.VMEM((NSUB, SUB), jnp.int32),
            pltpu.VMEM((NSUB, SUB), jnp.int32),
            pltpu.VMEM((NSUB, SUB), jnp.int32),
            pltpu.VMEM((NSUB, SUB), jnp.int32),
            pltpu.VMEM((NSUB, SUB), jnp.int32),
            pltpu.VMEM((NLIN, CHUNK), jnp.float32),
            pltpu.VMEM((NG, CHUNK), jnp.float32),
            pltpu.VMEM((NG, CHUNK), jnp.float32),
            pltpu.VMEM((NG, CHUNK, HALF), jnp.float32),
            pltpu.SemaphoreType.DMA((NLIN,)),
            pltpu.SemaphoreType.DMA((NG,)),
            pltpu.SemaphoreType.DMA((NG,)),
            pltpu.SemaphoreType.DMA((NG,)),
        ],
        compiler_params=pltpu.CompilerParams(use_tc_tiling_on_sc=False),
    )


@jax.jit
def kernel(edge_index, edge_weight, user_emb):
    row = edge_index[0].reshape(NS, EPT_RAW)
    col = edge_index[1].reshape(NS, EPT_RAW)
    w = edge_weight.reshape(NS, EPT_RAW)
    pad = EPT - EPT_RAW
    rowp = jnp.pad(row, ((0, 0), (0, pad))).reshape(-1)
    colp = jnp.pad(col, ((0, 0), (0, pad))).reshape(-1)
    wp = jnp.pad(w, ((0, 0), (0, pad))).reshape(-1)
    rowx = jnp.concatenate([rowp, rowp + N_USERS]).reshape(NC * R2D_TOTAL, SUB)
    col2 = colp.reshape(R2D_TOTAL, SUB)
    xh = user_emb.reshape(N_USERS, NC, HALF).transpose(1, 0, 2).reshape(
        NC * N_USERS, HALF)
    social = _make_kernel()(xh, rowx, col2, wp)
    return social.reshape(NC, N_USERS, HALF).transpose(1, 0, 2).reshape(
        N_USERS, EMB_DIM)
